# baseline (speedup 1.0000x reference)
import functools

import jax
import jax.numpy as jnp
from jax import lax
from jax.experimental import pallas as pl
from jax.experimental.pallas import tpu as pltpu
from jax.experimental.pallas import tpu_sc as plsc

F = 26
D = 16
B = 16384
FIELD = 40000
V = F * FIELD
NC, NS = 2, 16
NW = NC * NS
BPW = B // NW
CH = 64
NCH = BPW // CH
RPC = CH * F
JPC = RPC // 128
NIDXROW = BPW * F // 128

NTILE = V // 128
UPW = 64
NBUF = 4
LPW = V // NS


def _tr_body(x4_hbm, out_hbm, in0, in1, in2, in3, ot0, ot1, ot2, ot3,
             si0, si1, si2, si3, so0, so1, so2, so3):
    wid = lax.axis_index("s") * NC + lax.axis_index("c")
    ins = (in0, in1, in2, in3)
    ots = (ot0, ot1, ot2, ot3)
    sis = (si0, si1, si2, si3)
    sos = (so0, so1, so2, so3)
    io = lax.iota(jnp.int32, 16)
    gbase = (io // 8) * 4096 + (io % 8) * 128

    def t0_of(k):
        return jnp.minimum((wid * UPW + k) * 4, NTILE - 4)

    def fire_in(k, b):
        t0 = t0_of(k)
        for dh in range(2):
            pltpu.async_copy(
                x4_hbm.at[dh, pl.ds(t0 * 1024, 4096)],
                ins[b].at[pl.ds(dh * 4096, 4096)], sis[b])

    def wait_in(b):
        pltpu.make_async_copy(
            x4_hbm.at[0, pl.ds(0, 8192)], ins[b], sis[b]).wait()

    def wait_out(b):
        pltpu.make_async_copy(
            out_hbm.at[pl.ds(0, 8192)], ots[b], sos[b]).wait()

    def compute_and_out(k, b):
        def col(i, carry):
            for u in range(4):
                c = i * 4 + u
                cols = (io + c) & 127
                c16 = cols * 16
                for j in range(4):
                    v = plsc.load_gather(ins[b], [gbase + (cols + j * 1024)])
                    plsc.store_scatter(ots[b], [c16 + (io + j * 2048)], v)
            return carry

        lax.fori_loop(0, 32, col, 0)
        pltpu.async_copy(ots[b], out_hbm.at[pl.ds(t0_of(k) * 2048, 8192)],
                         sos[b])

    for b in range(NBUF):
        fire_in(b, b)
    for b in range(NBUF):
        wait_in(b)
        compute_and_out(b, b)
        fire_in(NBUF + b, b)

    def ring(kt, carry):
        k = kt * NBUF
        for j in range(NBUF):
            wait_in(j)
            wait_out(j)
            compute_and_out(k + j, j)

            @pl.when(k + j + NBUF < UPW)
            def _():
                fire_in(k + j + NBUF, j)
        return carry

    lax.fori_loop(1, UPW // NBUF, ring, 0)
    for b in range(NBUF):
        wait_out(b)


@jax.jit
def _tr_call(x4):
    mesh = plsc.VectorSubcoreMesh(
        core_axis_name="c", subcore_axis_name="s",
        num_cores=NC, num_subcores=NS)
    run = pl.kernel(
        _tr_body,
        out_type=jax.ShapeDtypeStruct((V * D,), jnp.float32),
        mesh=mesh,
        compiler_params=pltpu.CompilerParams(
            needs_layout_passes=False, use_tc_tiling_on_sc=False),
        scratch_types=(
            [pltpu.VMEM((8192,), jnp.float32)] * NBUF
            + [pltpu.VMEM((8192,), jnp.float32)] * NBUF
            + [pltpu.SemaphoreType.DMA] * (2 * NBUF)
        ),
    )
    return run(x4)


def _fm_body(idx_hbm, wlin_hbm, wfm_hbm, out_hbm,
             idx_v, lin_v, rows_v, rvec_v, out_v, sem0, sem1):
    cid = lax.axis_index("c")
    sid = lax.axis_index("s")
    wid = sid * NC + cid
    base = wid * BPW
    sems = (sem0, sem1)

    io = lax.iota(jnp.int32, 16)
    io16 = io * 16
    io26 = io * 26

    def stage_idx(c, buf):
        pltpu.sync_copy(idx_hbm.at[wid].at[pl.ds(c * JPC, JPC)],
                        idx_v.at[buf])

    def fire(c, buf):
        for j in range(JPC):
            idx_row = idx_v.at[buf, j]
            pltpu.async_copy(
                wfm_hbm.at[idx_row],
                rows_v.at[buf].at[pl.ds(j * 128, 128)],
                sems[buf])
            pltpu.async_copy(
                wlin_hbm.at[idx_row],
                lin_v.at[buf].at[pl.ds(j * 128, 128)],
                sems[buf])

    def drain(buf):
        pltpu.make_async_copy(
            wfm_hbm.at[pl.ds(0, RPC)], rows_v.at[buf], sems[buf]).wait()
        pltpu.make_async_copy(
            wlin_hbm.at[pl.ds(0, RPC)], lin_v.at[buf], sems[buf]).wait()

    def compute(c, buf):
        rows2 = rows_v.at[buf]
        for g in range(CH // 16):
            def sample_body(i, carry, _g=g):
                r0 = (_g * 16 + i) * F
                vs = [None, None, None, None]
                qs = [None, None, None, None]
                for f in range(F):
                    v = rows2[r0 + f, :]
                    a = f % 4
                    vs[a] = v if vs[a] is None else vs[a] + v
                    qs[a] = v * v if qs[a] is None else qs[a] + v * v
                s = (vs[0] + vs[1]) + (vs[2] + vs[3])
                q = (qs[0] + qs[1]) + (qs[2] + qs[3])
                rvec_v[pl.ds(i * 16, 16)] = s * s - q
                return carry

            lax.fori_loop(0, 16, sample_body, 0)

            fm0 = plsc.load_gather(rvec_v, [io16 + (io & 15)])
            fm1 = plsc.load_gather(rvec_v, [io16 + ((io + 1) & 15)])
            for dd in range(2, D, 2):
                fm0 = fm0 + plsc.load_gather(rvec_v, [io16 + ((io + dd) & 15)])
                fm1 = fm1 + plsc.load_gather(
                    rvec_v, [io16 + ((io + dd + 1) & 15)])
            fm = fm0 + fm1

            lv = lin_v.at[buf]
            li = io26 + g * 16 * F
            ln0 = plsc.load_gather(lv, [li])
            ln1 = plsc.load_gather(lv, [li + 1])
            for f in range(2, F, 2):
                ln0 = ln0 + plsc.load_gather(lv, [li + f])
                ln1 = ln1 + plsc.load_gather(lv, [li + f + 1])
            z = fm + (ln0 + ln1)

            y = 1.0 / (1.0 + jnp.exp(-z))
            out_v[pl.ds(c * CH + g * 16, 16)] = y

    stage_idx(0, 0)
    fire(0, 0)

    def outer(cp, carry):
        a = cp * 2
        stage_idx(a + 1, 1)
        fire(a + 1, 1)
        drain(0)
        compute(a, 0)

        @pl.when(a + 2 < NCH)
        def _():
            stage_idx(a + 2, 0)
            fire(a + 2, 0)

        drain(1)
        compute(a + 1, 1)
        return carry

    lax.fori_loop(0, NCH // 2, outer, 0)

    pltpu.sync_copy(out_v, out_hbm.at[pl.ds(base, BPW)])


@jax.jit
def _fm_call(idx, wlin, wfm):
    mesh = plsc.VectorSubcoreMesh(
        core_axis_name="c", subcore_axis_name="s",
        num_cores=NC, num_subcores=NS)
    run = pl.kernel(
        _fm_body,
        out_type=jax.ShapeDtypeStruct((B,), jnp.float32),
        mesh=mesh,
        compiler_params=pltpu.CompilerParams(
            needs_layout_passes=False, use_tc_tiling_on_sc=False),
        scratch_types=[
            pltpu.VMEM((2, JPC, 128), jnp.int32),
            pltpu.VMEM((2, RPC), jnp.float32),
            pltpu.VMEM((2, RPC, D), jnp.float32),
            pltpu.VMEM((256,), jnp.float32),
            pltpu.VMEM((BPW,), jnp.float32),
            pltpu.SemaphoreType.DMA,
            pltpu.SemaphoreType.DMA,
        ],
    )
    return run(idx, wlin, wfm)


def kernel(x, W_lin, W_fm):
    offs = jnp.arange(F, dtype=jnp.int32) * FIELD
    xi = x.astype(jnp.int32) + offs[None, :]
    idx = xi.reshape(NW, NIDXROW, 128)
    x4 = jnp.transpose(W_fm.T.reshape(2, 8, NTILE, 128),
                       (0, 2, 1, 3)).reshape(2, NTILE * 1024)
    wrow = _tr_call(x4).reshape(V, D)
    y = _fm_call(idx, W_lin.reshape(-1), wrow)
    return y.reshape(B, 1)


# speedup vs baseline: 1.0086x; 1.0086x over previous
import functools

import jax
import jax.numpy as jnp
from jax import lax
from jax.experimental import pallas as pl
from jax.experimental.pallas import tpu as pltpu
from jax.experimental.pallas import tpu_sc as plsc

F = 26
D = 16
B = 16384
FIELD = 40000
V = F * FIELD
NC, NS = 2, 16
NW = NC * NS
BPW = B // NW
CH = 64
NCH = BPW // CH
RPC = CH * F
JPC = RPC // 128
NIDXROW = BPW * F // 128

NTILE = V // 128
UPW = 128
NBUF = 4
LPW = V // NS


def _tr_body(x4_hbm, out_hbm, in0, in1, in2, in3, ot0, ot1, ot2, ot3,
             si0, si1, si2, si3, so0, so1, so2, so3):
    wid = lax.axis_index("s") * NC + lax.axis_index("c")
    ins = (in0, in1, in2, in3)
    ots = (ot0, ot1, ot2, ot3)
    sis = (si0, si1, si2, si3)
    sos = (so0, so1, so2, so3)
    io = lax.iota(jnp.int32, 16)
    io_hi = io // 8
    io_lo = io % 8

    def t0_of(k):
        return jnp.minimum((wid * UPW + k) * 2, NTILE - 2)

    def fire_in(k, b):
        t0 = t0_of(k)
        for dh in range(2):
            pltpu.async_copy(
                x4_hbm.at[dh, pl.ds(t0, 2)], ins[b].at[dh], sis[b])

    def wait_in(b):
        pltpu.make_async_copy(
            x4_hbm.at[0, pl.ds(0, 2)], ins[b].at[0], sis[b]).wait()
        pltpu.make_async_copy(
            x4_hbm.at[0, pl.ds(0, 2)], ins[b].at[1], sis[b]).wait()

    def wait_out(b):
        pltpu.make_async_copy(
            out_hbm.at[pl.ds(0, 256)], ots[b], sos[b]).wait()

    def compute_and_out(k, b):
        def col(i, carry):
            for u in range(4):
                c = i * 4 + u
                cols = (io + c) & 127
                for j in range(2):
                    v = plsc.load_gather(
                        ins[b], [io_hi, jnp.full((16,), j, jnp.int32),
                                 io_lo, cols])
                    plsc.store_scatter(ots[b], [cols + j * 128, io], v)
            return carry

        lax.fori_loop(0, 32, col, 0)
        pltpu.async_copy(ots[b], out_hbm.at[pl.ds(t0_of(k) * 128, 256)],
                         sos[b])

    for b in range(NBUF):
        fire_in(b, b)
    for b in range(NBUF):
        wait_in(b)
        compute_and_out(b, b)
        fire_in(NBUF + b, b)

    def ring(kt, carry):
        k = kt * NBUF
        for j in range(NBUF):
            wait_in(j)
            wait_out(j)
            compute_and_out(k + j, j)

            @pl.when(k + j + NBUF < UPW)
            def _():
                fire_in(k + j + NBUF, j)
        return carry

    lax.fori_loop(1, UPW // NBUF, ring, 0)
    for b in range(NBUF):
        wait_out(b)


@jax.jit
def _tr_call(x4):
    mesh = plsc.VectorSubcoreMesh(
        core_axis_name="c", subcore_axis_name="s",
        num_cores=NC, num_subcores=NS)
    run = pl.kernel(
        _tr_body,
        out_type=jax.ShapeDtypeStruct((V, D), jnp.float32),
        mesh=mesh,
        compiler_params=pltpu.CompilerParams(
            needs_layout_passes=False, use_tc_tiling_on_sc=False),
        scratch_types=(
            [pltpu.VMEM((2, 2, 8, 128), jnp.float32)] * NBUF
            + [pltpu.VMEM((256, D), jnp.float32)] * NBUF
            + [pltpu.SemaphoreType.DMA] * (2 * NBUF)
        ),
    )
    return run(x4)


def _fm_body(idx_hbm, wlin_hbm, wfm_hbm, out_hbm,
             idx_v, lin_v, rows_v, rvec_v, out_v, sem0, sem1):
    cid = lax.axis_index("c")
    sid = lax.axis_index("s")
    wid = sid * NC + cid
    base = wid * BPW
    sems = (sem0, sem1)

    io = lax.iota(jnp.int32, 16)
    io16 = io * 16
    io26 = io * 26

    def stage_idx(c, buf):
        pltpu.sync_copy(idx_hbm.at[wid].at[pl.ds(c * JPC, JPC)],
                        idx_v.at[buf])

    def fire(c, buf):
        for j in range(JPC):
            idx_row = idx_v.at[buf, j]
            pltpu.async_copy(
                wfm_hbm.at[idx_row],
                rows_v.at[buf].at[pl.ds(j * 128, 128)],
                sems[buf])
            pltpu.async_copy(
                wlin_hbm.at[idx_row],
                lin_v.at[buf].at[pl.ds(j * 128, 128)],
                sems[buf])

    def drain(buf):
        pltpu.make_async_copy(
            wfm_hbm.at[pl.ds(0, RPC)], rows_v.at[buf], sems[buf]).wait()
        pltpu.make_async_copy(
            wlin_hbm.at[pl.ds(0, RPC)], lin_v.at[buf], sems[buf]).wait()

    def compute(c, buf):
        rows2 = rows_v.at[buf]
        for g in range(CH // 16):
            def sample_body(i, carry, _g=g):
                r0 = (_g * 16 + i) * F
                vs = [None, None, None, None]
                qs = [None, None, None, None]
                for f in range(F):
                    v = rows2[r0 + f, :]
                    a = f % 4
                    vs[a] = v if vs[a] is None else vs[a] + v
                    qs[a] = v * v if qs[a] is None else qs[a] + v * v
                s = (vs[0] + vs[1]) + (vs[2] + vs[3])
                q = (qs[0] + qs[1]) + (qs[2] + qs[3])
                rvec_v[pl.ds(i * 16, 16)] = s * s - q
                return carry

            lax.fori_loop(0, 16, sample_body, 0)

            fm0 = plsc.load_gather(rvec_v, [io16 + (io & 15)])
            fm1 = plsc.load_gather(rvec_v, [io16 + ((io + 1) & 15)])
            for dd in range(2, D, 2):
                fm0 = fm0 + plsc.load_gather(rvec_v, [io16 + ((io + dd) & 15)])
                fm1 = fm1 + plsc.load_gather(
                    rvec_v, [io16 + ((io + dd + 1) & 15)])
            fm = fm0 + fm1

            lv = lin_v.at[buf]
            li = io26 + g * 16 * F
            ln0 = plsc.load_gather(lv, [li])
            ln1 = plsc.load_gather(lv, [li + 1])
            for f in range(2, F, 2):
                ln0 = ln0 + plsc.load_gather(lv, [li + f])
                ln1 = ln1 + plsc.load_gather(lv, [li + f + 1])
            z = fm + (ln0 + ln1)

            y = 1.0 / (1.0 + jnp.exp(-z))
            out_v[pl.ds(c * CH + g * 16, 16)] = y

    stage_idx(0, 0)
    fire(0, 0)

    def outer(cp, carry):
        a = cp * 2
        stage_idx(a + 1, 1)
        fire(a + 1, 1)
        drain(0)
        compute(a, 0)

        @pl.when(a + 2 < NCH)
        def _():
            stage_idx(a + 2, 0)
            fire(a + 2, 0)

        drain(1)
        compute(a + 1, 1)
        return carry

    lax.fori_loop(0, NCH // 2, outer, 0)

    pltpu.sync_copy(out_v, out_hbm.at[pl.ds(base, BPW)])


@jax.jit
def _fm_call(idx, wlin, wfm):
    mesh = plsc.VectorSubcoreMesh(
        core_axis_name="c", subcore_axis_name="s",
        num_cores=NC, num_subcores=NS)
    run = pl.kernel(
        _fm_body,
        out_type=jax.ShapeDtypeStruct((B,), jnp.float32),
        mesh=mesh,
        compiler_params=pltpu.CompilerParams(
            needs_layout_passes=False, use_tc_tiling_on_sc=False),
        scratch_types=[
            pltpu.VMEM((2, JPC, 128), jnp.int32),
            pltpu.VMEM((2, RPC), jnp.float32),
            pltpu.VMEM((2, RPC, D), jnp.float32),
            pltpu.VMEM((256,), jnp.float32),
            pltpu.VMEM((BPW,), jnp.float32),
            pltpu.SemaphoreType.DMA,
            pltpu.SemaphoreType.DMA,
        ],
    )
    return run(idx, wlin, wfm)


def kernel(x, W_lin, W_fm):
    offs = jnp.arange(F, dtype=jnp.int32) * FIELD
    xi = x.astype(jnp.int32) + offs[None, :]
    idx = xi.reshape(NW, NIDXROW, 128)
    x4 = jnp.transpose(W_fm.T.reshape(2, 8, NTILE, 128), (0, 2, 1, 3))
    wrow = _tr_call(x4)
    y = _fm_call(idx, W_lin.reshape(-1), wrow)
    return y.reshape(B, 1)


# speedup vs baseline: 1.0094x; 1.0008x over previous
import jax
import jax.numpy as jnp
from jax import lax
from jax.experimental import pallas as pl
from jax.experimental.pallas import tpu as pltpu
from jax.experimental.pallas import tpu_sc as plsc

F = 26
D = 16
B = 16384
FIELD = 40000
V = F * FIELD
NC, NS = 2, 16
NW = NC * NS
BPW = B // NW
CH = 64
NCH = BPW // CH
RPC = CH * F
JPC = RPC // 128
NIDXROW = BPW * F // 128

NTILE = V // 128
UPW = 128
NBUF = 4


def _tr_body(x4_hbm, out_hbm, in0, in1, in2, in3, ot0, ot1, ot2, ot3,
             si0, si1, si2, si3, so0, so1, so2, so3):
    wid = lax.axis_index("s") * NC + lax.axis_index("c")
    ins = (in0, in1, in2, in3)
    ots = (ot0, ot1, ot2, ot3)
    sis = (si0, si1, si2, si3)
    sos = (so0, so1, so2, so3)
    io = lax.iota(jnp.int32, 16)
    io_hi = io // 8
    io_lo = io % 8

    def t0_of(k):
        return jnp.minimum((wid * UPW + k) * 2, NTILE - 2)

    def fire_in(k, b):
        t0 = t0_of(k)
        for dh in range(2):
            pltpu.async_copy(
                x4_hbm.at[dh, pl.ds(t0, 2)], ins[b].at[dh], sis[b])

    def wait_in(b):
        pltpu.make_async_copy(
            x4_hbm.at[0, pl.ds(0, 2)], ins[b].at[0], sis[b]).wait()
        pltpu.make_async_copy(
            x4_hbm.at[0, pl.ds(0, 2)], ins[b].at[1], sis[b]).wait()

    def wait_out(b):
        pltpu.make_async_copy(
            out_hbm.at[pl.ds(0, 256)], ots[b], sos[b]).wait()

    def compute_and_out(k, b):
        def col(i, carry):
            for u in range(4):
                c = i * 4 + u
                cols = (io + c) & 127
                for j in range(2):
                    v = plsc.load_gather(
                        ins[b], [io_hi, jnp.full((16,), j, jnp.int32),
                                 io_lo, cols])
                    plsc.store_scatter(ots[b], [cols + j * 128, io], v)
            return carry

        lax.fori_loop(0, 32, col, 0)
        pltpu.async_copy(ots[b], out_hbm.at[pl.ds(t0_of(k) * 128, 256)],
                         sos[b])

    for b in range(NBUF):
        fire_in(b, b)
    for b in range(NBUF):
        wait_in(b)
        compute_and_out(b, b)
        fire_in(NBUF + b, b)

    def ring(kt, carry):
        k = kt * NBUF
        for j in range(NBUF):
            wait_in(j)
            wait_out(j)
            compute_and_out(k + j, j)

            @pl.when(k + j + NBUF < UPW)
            def _():
                fire_in(k + j + NBUF, j)
        return carry

    lax.fori_loop(1, UPW // NBUF, ring, 0)
    for b in range(NBUF):
        wait_out(b)


@jax.jit
def _tr_call(x4):
    mesh = plsc.VectorSubcoreMesh(
        core_axis_name="c", subcore_axis_name="s",
        num_cores=NC, num_subcores=NS)
    run = pl.kernel(
        _tr_body,
        out_type=jax.ShapeDtypeStruct((V, D), jnp.float32),
        mesh=mesh,
        compiler_params=pltpu.CompilerParams(
            needs_layout_passes=False, use_tc_tiling_on_sc=False),
        scratch_types=(
            [pltpu.VMEM((2, 2, 8, 128), jnp.float32)] * NBUF
            + [pltpu.VMEM((256, D), jnp.float32)] * NBUF
            + [pltpu.SemaphoreType.DMA] * (2 * NBUF)
        ),
    )
    return run(x4)


def _fm_body(idx_hbm, wlin_hbm, wfm_hbm, out_hbm,
             idx_v, lin_v, rows_v, rvec_v, out_v, sem0, sem1):
    cid = lax.axis_index("c")
    sid = lax.axis_index("s")
    wid = sid * NC + cid
    base = wid * BPW
    sems = (sem0, sem1)

    io = lax.iota(jnp.int32, 16)
    io16 = io * 16
    io26 = io * 26

    def stage_idx(c, buf):
        pltpu.sync_copy(idx_hbm.at[wid].at[pl.ds(c * JPC, JPC)],
                        idx_v.at[buf])

    def fire(c, buf):
        for j in range(JPC):
            idx_row = idx_v.at[buf, j]
            pltpu.async_copy(
                wfm_hbm.at[idx_row],
                rows_v.at[buf].at[pl.ds(j * 128, 128)],
                sems[buf])
            pltpu.async_copy(
                wlin_hbm.at[idx_row],
                lin_v.at[buf].at[pl.ds(j * 128, 128)],
                sems[buf])

    def drain(buf):
        pltpu.make_async_copy(
            wfm_hbm.at[pl.ds(0, RPC)], rows_v.at[buf], sems[buf]).wait()
        pltpu.make_async_copy(
            wlin_hbm.at[pl.ds(0, RPC)], lin_v.at[buf], sems[buf]).wait()

    def compute(c, buf):
        rows2 = rows_v.at[buf]
        for g in range(CH // 16):
            def sample_body(i, carry, _g=g):
                r0 = (_g * 16 + i) * F
                vs = [None, None, None, None]
                qs = [None, None, None, None]
                for f in range(F):
                    v = rows2[r0 + f, :]
                    a = f % 4
                    vs[a] = v if vs[a] is None else vs[a] + v
                    qs[a] = v * v if qs[a] is None else qs[a] + v * v
                s = (vs[0] + vs[1]) + (vs[2] + vs[3])
                q = (qs[0] + qs[1]) + (qs[2] + qs[3])
                rvec_v[pl.ds(i * 16, 16)] = s * s - q
                return carry

            lax.fori_loop(0, 16, sample_body, 0)

            fm0 = plsc.load_gather(rvec_v, [io16 + (io & 15)])
            fm1 = plsc.load_gather(rvec_v, [io16 + ((io + 1) & 15)])
            for dd in range(2, D, 2):
                fm0 = fm0 + plsc.load_gather(rvec_v, [io16 + ((io + dd) & 15)])
                fm1 = fm1 + plsc.load_gather(
                    rvec_v, [io16 + ((io + dd + 1) & 15)])
            fm = fm0 + fm1

            lv = lin_v.at[buf]
            li = io26 + g * 16 * F
            ln0 = plsc.load_gather(lv, [li])
            ln1 = plsc.load_gather(lv, [li + 1])
            for f in range(2, F, 2):
                ln0 = ln0 + plsc.load_gather(lv, [li + f])
                ln1 = ln1 + plsc.load_gather(lv, [li + f + 1])
            z = fm + (ln0 + ln1)

            y = 1.0 / (1.0 + jnp.exp(-z))
            out_v[pl.ds(c * CH + g * 16, 16)] = y

    stage_idx(0, 0)
    fire(0, 0)

    def outer(cp, carry):
        a = cp * 2
        stage_idx(a + 1, 1)
        fire(a + 1, 1)
        drain(0)
        compute(a, 0)

        @pl.when(a + 2 < NCH)
        def _():
            stage_idx(a + 2, 0)
            fire(a + 2, 0)

        drain(1)
        compute(a + 1, 1)
        return carry

    lax.fori_loop(0, NCH // 2, outer, 0)

    pltpu.sync_copy(out_v, out_hbm.at[pl.ds(base, BPW)])


@jax.jit
def _fm_call(idx, wlin, wfm):
    mesh = plsc.VectorSubcoreMesh(
        core_axis_name="c", subcore_axis_name="s",
        num_cores=NC, num_subcores=NS)
    run = pl.kernel(
        _fm_body,
        out_type=jax.ShapeDtypeStruct((B,), jnp.float32),
        mesh=mesh,
        compiler_params=pltpu.CompilerParams(
            needs_layout_passes=False, use_tc_tiling_on_sc=False),
        scratch_types=[
            pltpu.VMEM((2, JPC, 128), jnp.int32),
            pltpu.VMEM((2, RPC), jnp.float32),
            pltpu.VMEM((2, RPC, D), jnp.float32),
            pltpu.VMEM((256,), jnp.float32),
            pltpu.VMEM((BPW,), jnp.float32),
            pltpu.SemaphoreType.DMA,
            pltpu.SemaphoreType.DMA,
        ],
    )
    return run(idx, wlin, wfm)


def kernel(x, W_lin, W_fm):
    offs = jnp.arange(F, dtype=jnp.int32) * FIELD
    xi = x.astype(jnp.int32) + offs[None, :]
    idx = xi.reshape(NW, NIDXROW, 128)
    x4 = jnp.transpose(W_fm.T.reshape(2, 8, NTILE, 128), (0, 2, 1, 3))
    wrow = _tr_call(x4)
    y = _fm_call(idx, W_lin.reshape(-1), wrow)
    return y.reshape(B, 1)
